# baseline (speedup 1.0000x reference)
import functools

import jax
import jax.numpy as jnp
from jax import lax
from jax.experimental import pallas as pl
from jax.experimental.pallas import tpu as pltpu
from jax.experimental.pallas import tpu_sc as plsc

_IN_DIM = 256
_OUT_DIM = 512
_MEM_LEN = 100000
_BETA = 1.0
_BLK_B = 5000

_NC = 2
_NS = 16
_NW = _NC * _NS
_TILES = _MEM_LEN // 8
_TQ = _TILES // _NW
_TR = _TILES - _TQ * _NW
_CA = 240
_NCA = 8 * _TQ // _CA


def _sc_pass_a_body(md_hbm, out_hbm, ps_hbm, buf0, buf1, st,
                    sin0, sin1, sout0, sout1, sem_x):
    wid = lax.axis_index("s") * _NC + lax.axis_index("c")
    base = 8 * (wid * _TQ + jnp.minimum(wid, _TR))
    bufs = (buf0, buf1)
    sins = (sin0, sin1)
    souts = (sout0, sout1)
    zero = jnp.zeros((16,), jnp.float32)
    carry = tuple([zero] * 32)

    def _row_body(buf):
        def row_body(r, c):
            out = list(c)
            for j in range(16):
                v = buf[r, pl.ds(j * 16, 16)]
                out[j] = c[j] + v
                out[16 + j] = c[16 + j] + v * v
            return tuple(out)
        return row_body

    n_extra = jnp.where(wid < _TR, 8, 0)

    @pl.when(wid < _TR)
    def _():
        extra = base + 8 * _TQ
        pltpu.async_copy(
            md_hbm.at[pl.ds(extra, 8)], buf1.at[pl.ds(0, 8)], sem_x).wait()
        pltpu.async_copy(
            buf1.at[pl.ds(0, 8)], out_hbm.at[pl.ds(extra, 8)], sem_x).wait()

    carry = lax.fori_loop(0, n_extra, _row_body(buf1), carry)

    in_h = [None, None]
    out_h = [None, None]
    in_h[0] = pltpu.async_copy(md_hbm.at[pl.ds(base, _CA)], buf0, sin0)
    for k in range(_NCA):
        b = k % 2
        nb = (k + 1) % 2
        if k + 1 < _NCA:
            if k >= 1:
                out_h[nb].wait()
            in_h[nb] = pltpu.async_copy(
                md_hbm.at[pl.ds(base + (k + 1) * _CA, _CA)], bufs[nb],
                sins[nb])
        in_h[b].wait()
        out_h[b] = pltpu.async_copy(
            bufs[b], out_hbm.at[pl.ds(base + k * _CA, _CA)], souts[b])
        carry = lax.fori_loop(0, _CA, _row_body(bufs[b]), carry, unroll=4)

    for j in range(16):
        st[0, pl.ds(j * 16, 16)] = carry[j]
        st[1, pl.ds(j * 16, 16)] = carry[16 + j]
    pltpu.sync_copy(st, ps_hbm.at[wid])
    out_h[(_NCA - 2) % 2].wait()
    out_h[(_NCA - 1) % 2].wait()


_sc_pass_a = functools.partial(
    pl.kernel,
    mesh=plsc.VectorSubcoreMesh(core_axis_name="c", subcore_axis_name="s"),
    out_type=[
        jax.ShapeDtypeStruct((_MEM_LEN, _IN_DIM), jnp.float32),
        jax.ShapeDtypeStruct((_NW, 8, _IN_DIM), jnp.float32),
    ],
    scratch_types=[
        pltpu.VMEM((_CA, _IN_DIM), jnp.float32),
        pltpu.VMEM((_CA, _IN_DIM), jnp.float32),
        pltpu.VMEM((8, _IN_DIM), jnp.float32),
        pltpu.SemaphoreType.DMA,
        pltpu.SemaphoreType.DMA,
        pltpu.SemaphoreType.DMA,
        pltpu.SemaphoreType.DMA,
        pltpu.SemaphoreType.DMA,
    ],
)(_sc_pass_a_body)


def _mlp_body(x_ref, ps_ref, w1, b1, w2, b2, w3, b3, enc_ref):
    n = jnp.float32(_MEM_LEN)
    ps = ps_ref[...]
    s = jnp.sum(ps[:, 0, :], axis=0, keepdims=True)
    q = jnp.sum(ps[:, 1, :], axis=0, keepdims=True)
    mean = s / n
    var = (q - s * (s / n)) / (n - 1.0)
    std = jnp.sqrt(var)
    xn = (x_ref[...] - mean) / std
    xn = jnp.where(std == 0.0, 0.0, xn)
    h1 = jnp.maximum(
        jnp.dot(xn, w1[...], preferred_element_type=jnp.float32) + b1[...], 0.0)
    h2 = jnp.maximum(
        jnp.dot(h1, w2[...], preferred_element_type=jnp.float32) + b2[...], 0.0)
    enc_ref[...] = jnp.tanh(
        jnp.dot(h2, w3[...], preferred_element_type=jnp.float32) + b3[...])


def _pass_b_body(mem_ref, enc_ref, x_ref, md_in, out_ref, loss_ref, md_out,
                 min_s):
    i = pl.program_id(0)
    blk = mem_ref[...]
    out_ref[...] = blk
    m = jnp.min(jnp.sum(jnp.abs(blk - enc_ref[...]), axis=1))

    @pl.when(i == 0)
    def _():
        min_s[0] = m

    @pl.when(i > 0)
    def _():
        min_s[0] = jnp.minimum(min_s[0], m)

    @pl.when(i == pl.num_programs(0) - 1)
    def _():
        loss = min_s[0]
        loss_ref[0, 0] = loss
        md_out[...] = md_in[...]

        @pl.when(loss <= _BETA)
        def _():
            out_ref[0:1, :] = enc_ref[...]
            md_out[0:1, :] = x_ref[...]


def kernel(x, mem_data, memory, W1, b1, W2, b2, W3, b3):
    f32 = jnp.float32
    W1p = jnp.pad(W1, ((0, 0), (0, 12)))
    b1p = jnp.pad(b1, (0, 12)).reshape(1, 512)
    W2p = jnp.pad(W2, ((0, 12), (0, 24)))
    b2p = jnp.pad(b2, (0, 24)).reshape(1, 1024)
    W3p = jnp.pad(W3, ((0, 24), (0, 0)))
    b3p = b3.reshape(1, 512)

    new_mem_data, part_stats = _sc_pass_a(mem_data)

    enc = pl.pallas_call(
        _mlp_body,
        out_shape=jax.ShapeDtypeStruct((1, _OUT_DIM), f32),
    )(x, part_stats, W1p, b1p, W2p, b2p, W3p, b3p)

    nb = _MEM_LEN // _BLK_B
    new_memory, loss11, new_mem_data = pl.pallas_call(
        _pass_b_body,
        grid=(nb,),
        in_specs=[
            pl.BlockSpec((_BLK_B, _OUT_DIM), lambda i, nb=nb: (nb - 1 - i, 0)),
            pl.BlockSpec((1, _OUT_DIM), lambda i: (0, 0)),
            pl.BlockSpec((1, _IN_DIM), lambda i: (0, 0)),
            pl.BlockSpec((8, _IN_DIM), lambda i: (0, 0)),
        ],
        out_specs=[
            pl.BlockSpec((_BLK_B, _OUT_DIM), lambda i, nb=nb: (nb - 1 - i, 0)),
            pl.BlockSpec(memory_space=pltpu.SMEM),
            pl.BlockSpec((8, _IN_DIM), lambda i: (0, 0)),
        ],
        out_shape=[
            jax.ShapeDtypeStruct((_MEM_LEN, _OUT_DIM), f32),
            jax.ShapeDtypeStruct((1, 1), f32),
            jax.ShapeDtypeStruct((_MEM_LEN, _IN_DIM), f32),
        ],
        scratch_shapes=[pltpu.SMEM((1,), f32)],
        input_output_aliases={3: 2},
    )(memory, enc, x, new_mem_data)

    return loss11[0, 0], new_memory, new_mem_data


# speedup vs baseline: 1.0045x; 1.0045x over previous
import functools

import jax
import jax.numpy as jnp
from jax import lax
from jax.experimental import pallas as pl
from jax.experimental.pallas import tpu as pltpu
from jax.experimental.pallas import tpu_sc as plsc

_IN_DIM = 256
_OUT_DIM = 512
_MEM_LEN = 100000
_BETA = 1.0
_BLK_B = 5000

_NC = 2
_NS = 16
_NW = _NC * _NS
_TILES = _MEM_LEN // 8
_TQ = _TILES // _NW
_TR = _TILES - _TQ * _NW
_CA = 240
_NCA = 8 * _TQ // _CA


def _sc_pass_a_body(md_hbm, out_hbm, ps_hbm, buf0, buf1, st,
                    sin0, sin1, sout0, sout1, sem_x):
    wid = lax.axis_index("s") * _NC + lax.axis_index("c")
    base = 8 * (wid * _TQ + jnp.minimum(wid, _TR))
    bufs = (buf0, buf1)
    sins = (sin0, sin1)
    souts = (sout0, sout1)
    zero = jnp.zeros((16,), jnp.float32)
    carry = tuple([zero] * 32)

    def _row_body(buf):
        def row_body(r, c):
            out = list(c)
            for j in range(16):
                v = buf[r, pl.ds(j * 16, 16)]
                out[j] = c[j] + v
                out[16 + j] = c[16 + j] + v * v
            return tuple(out)
        return row_body

    n_extra = jnp.where(wid < _TR, 8, 0)

    @pl.when(wid < _TR)
    def _():
        extra = base + 8 * _TQ
        pltpu.async_copy(
            md_hbm.at[pl.ds(extra, 8)], buf1.at[pl.ds(0, 8)], sem_x).wait()
        pltpu.async_copy(
            buf1.at[pl.ds(0, 8)], out_hbm.at[pl.ds(extra, 8)], sem_x).wait()

    carry = lax.fori_loop(0, n_extra, _row_body(buf1), carry)

    in_h = [None, None]
    out_h = [None, None]
    in_h[0] = pltpu.async_copy(md_hbm.at[pl.ds(base, _CA)], buf0, sin0)
    for k in range(_NCA):
        b = k % 2
        nb = (k + 1) % 2
        if k + 1 < _NCA:
            if k >= 1:
                out_h[nb].wait()
            in_h[nb] = pltpu.async_copy(
                md_hbm.at[pl.ds(base + (k + 1) * _CA, _CA)], bufs[nb],
                sins[nb])
        in_h[b].wait()
        out_h[b] = pltpu.async_copy(
            bufs[b], out_hbm.at[pl.ds(base + k * _CA, _CA)], souts[b])
        carry = lax.fori_loop(0, _CA, _row_body(bufs[b]), carry, unroll=4)

    for j in range(16):
        st[0, pl.ds(j * 16, 16)] = carry[j]
        st[1, pl.ds(j * 16, 16)] = carry[16 + j]
    pltpu.sync_copy(st, ps_hbm.at[wid])
    out_h[(_NCA - 2) % 2].wait()
    out_h[(_NCA - 1) % 2].wait()


_sc_pass_a = functools.partial(
    pl.kernel,
    mesh=plsc.VectorSubcoreMesh(core_axis_name="c", subcore_axis_name="s"),
    out_type=[
        jax.ShapeDtypeStruct((_MEM_LEN, _IN_DIM), jnp.float32),
        jax.ShapeDtypeStruct((_NW, 8, _IN_DIM), jnp.float32),
    ],
    scratch_types=[
        pltpu.VMEM((_CA, _IN_DIM), jnp.float32),
        pltpu.VMEM((_CA, _IN_DIM), jnp.float32),
        pltpu.VMEM((8, _IN_DIM), jnp.float32),
        pltpu.SemaphoreType.DMA,
        pltpu.SemaphoreType.DMA,
        pltpu.SemaphoreType.DMA,
        pltpu.SemaphoreType.DMA,
        pltpu.SemaphoreType.DMA,
    ],
)(_sc_pass_a_body)


def _pass_b_body(mem_ref, ps_ref, x_ref, md_in, w1, b1, w2, b2, w3, b3,
                 out_ref, loss_ref, md_out, enc_s, min_s):
    i = pl.program_id(0)

    @pl.when(i == 0)
    def _():
        n = jnp.float32(_MEM_LEN)
        ps = ps_ref[...]
        s = jnp.sum(ps[:, 0, :], axis=0, keepdims=True)
        q = jnp.sum(ps[:, 1, :], axis=0, keepdims=True)
        mean = s / n
        var = (q - s * (s / n)) / (n - 1.0)
        std = jnp.sqrt(var)
        xn = (x_ref[...] - mean) / std
        xn = jnp.where(std == 0.0, 0.0, xn)
        h1 = jnp.maximum(
            jnp.dot(xn, w1[...], preferred_element_type=jnp.float32)
            + b1[...], 0.0)
        h2 = jnp.maximum(
            jnp.dot(h1, w2[...], preferred_element_type=jnp.float32)
            + b2[...], 0.0)
        enc_s[...] = jnp.tanh(
            jnp.dot(h2, w3[...], preferred_element_type=jnp.float32)
            + b3[...])

    @pl.when(i > 0)
    def _():
        blk = mem_ref[...]
        out_ref[...] = blk
        m = jnp.min(jnp.sum(jnp.abs(blk - enc_s[...]), axis=1))

        @pl.when(i == 1)
        def _():
            min_s[0] = m

        @pl.when(i > 1)
        def _():
            min_s[0] = jnp.minimum(min_s[0], m)

        @pl.when(i == pl.num_programs(0) - 1)
        def _():
            loss = min_s[0]
            loss_ref[0, 0] = loss
            md_out[...] = md_in[...]

            @pl.when(loss <= _BETA)
            def _():
                out_ref[0:1, :] = enc_s[...]
                md_out[0:1, :] = x_ref[...]


def kernel(x, mem_data, memory, W1, b1, W2, b2, W3, b3):
    f32 = jnp.float32
    W1p = jnp.pad(W1, ((0, 0), (0, 12)))
    b1p = jnp.pad(b1, (0, 12)).reshape(1, 512)
    W2p = jnp.pad(W2, ((0, 12), (0, 24)))
    b2p = jnp.pad(b2, (0, 24)).reshape(1, 1024)
    W3p = jnp.pad(W3, ((0, 24), (0, 0)))
    b3p = b3.reshape(1, 512)

    new_mem_data, part_stats = _sc_pass_a(mem_data)

    nb = _MEM_LEN // _BLK_B
    _mem_ix = lambda i, nb=nb: (nb - jnp.maximum(i, 1), 0)
    new_memory, loss11, new_mem_data = pl.pallas_call(
        _pass_b_body,
        grid=(nb + 1,),
        in_specs=[
            pl.BlockSpec((_BLK_B, _OUT_DIM), _mem_ix),
            pl.BlockSpec((_NW, 8, _IN_DIM), lambda i: (0, 0, 0)),
            pl.BlockSpec((1, _IN_DIM), lambda i: (0, 0)),
            pl.BlockSpec((8, _IN_DIM), lambda i: (0, 0)),
            pl.BlockSpec((_IN_DIM, 512), lambda i: (0, 0)),
            pl.BlockSpec((1, 512), lambda i: (0, 0)),
            pl.BlockSpec((512, 1024), lambda i: (0, 0)),
            pl.BlockSpec((1, 1024), lambda i: (0, 0)),
            pl.BlockSpec((1024, 512), lambda i: (0, 0)),
            pl.BlockSpec((1, 512), lambda i: (0, 0)),
        ],
        out_specs=[
            pl.BlockSpec((_BLK_B, _OUT_DIM), _mem_ix),
            pl.BlockSpec(memory_space=pltpu.SMEM),
            pl.BlockSpec((8, _IN_DIM), lambda i: (0, 0)),
        ],
        out_shape=[
            jax.ShapeDtypeStruct((_MEM_LEN, _OUT_DIM), f32),
            jax.ShapeDtypeStruct((1, 1), f32),
            jax.ShapeDtypeStruct((_MEM_LEN, _IN_DIM), f32),
        ],
        scratch_shapes=[
            pltpu.VMEM((1, _OUT_DIM), f32),
            pltpu.SMEM((1,), f32),
        ],
        input_output_aliases={3: 2},
    )(memory, part_stats, x, new_mem_data, W1p, b1p, W2p, b2p, W3p, b3p)

    return loss11[0, 0], new_memory, new_mem_data
